# baseline (speedup 1.0000x reference)
import jax
import jax.numpy as jnp
from jax.experimental import pallas as pl
from jax.experimental.pallas import tpu as pltpu
from functools import partial

_N = 50000
_E = 800000
_D = 128
_HID = 64
_H = 8
_HD = 8
_ROWS = 2000
_C = 6400


def _k1_body(x_ref, w_ref, asf_ref, adf_ref, out_ref):
    z = jnp.dot(x_ref[...], w_ref[...], preferred_element_type=jnp.float32)
    lane = jax.lax.broadcasted_iota(jnp.int32, (_HID, _H), 0)
    head = jax.lax.broadcasted_iota(jnp.int32, (_HID, _H), 1)
    g = ((lane // _HD) == head).astype(jnp.float32)
    es = jnp.dot(z * asf_ref[...], g, preferred_element_type=jnp.float32)
    ed = jnp.dot(z * adf_ref[...], g, preferred_element_type=jnp.float32)
    out_ref[...] = jnp.concatenate([z, es, ed], axis=1)


def _leaky(v):
    return jnp.where(v >= 0, v, 0.2 * v)


def _k2_body(src_ref, dst_ref, p_ref, q_ref):
    p = pl.program_id(0)
    c = pl.program_id(1)

    @pl.when(jnp.logical_and(p == 0, c == 0))
    def _init():
        q_ref[:, 0:64] = jnp.zeros((_N, 64), jnp.float32)
        q_ref[:, 64:72] = jnp.full((_N, _H), -jnp.inf, jnp.float32)
        q_ref[:, 72:80] = jnp.zeros((_N, _H), jnp.float32)

    def _e8(s, d):
        es = p_ref[pl.ds(s, 1), 64:72]
        ed = p_ref[pl.ds(d, 1), 72:80]
        return _leaky(es + ed)

    def body_max(j, _):
        s = src_ref[0, j]
        d = dst_ref[0, j]
        e8 = _e8(s, d)
        cur = q_ref[pl.ds(d, 1), 64:72]
        q_ref[pl.ds(d, 1), 64:72] = jnp.maximum(cur, e8)
        return 0

    def body_den(j, _):
        s = src_ref[0, j]
        d = dst_ref[0, j]
        e8 = _e8(s, d)
        m = q_ref[pl.ds(d, 1), 64:72]
        ee = jnp.exp(e8 - m)
        cur = q_ref[pl.ds(d, 1), 72:80]
        q_ref[pl.ds(d, 1), 72:80] = cur + ee
        return 0

    lane = jax.lax.broadcasted_iota(jnp.int32, (_H, _HID), 1)
    head = jax.lax.broadcasted_iota(jnp.int32, (_H, _HID), 0)
    rep = ((lane // _HD) == head).astype(jnp.float32)

    def body_out(j, _):
        s = src_ref[0, j]
        d = dst_ref[0, j]
        e8 = _e8(s, d)
        m = q_ref[pl.ds(d, 1), 64:72]
        dn = q_ref[pl.ds(d, 1), 72:80]
        alpha = jnp.exp(e8 - m) / (dn + 1e-16)
        a64 = jnp.dot(alpha, rep, preferred_element_type=jnp.float32)
        zs = p_ref[pl.ds(s, 1), 0:64]
        cur = q_ref[pl.ds(d, 1), 0:64]
        q_ref[pl.ds(d, 1), 0:64] = cur + zs * a64
        return 0

    @pl.when(p == 0)
    def _a():
        jax.lax.fori_loop(0, _C, body_max, 0)

    @pl.when(p == 1)
    def _b():
        jax.lax.fori_loop(0, _C, body_den, 0)

    @pl.when(p == 2)
    def _c():
        jax.lax.fori_loop(0, _C, body_out, 0)


def _ln(v, g, b):
    m = jnp.mean(v, axis=-1, keepdims=True)
    var = jnp.mean((v - m) * (v - m), axis=-1, keepdims=True)
    return (v - m) * jax.lax.rsqrt(var + 1e-5) * g + b


def _k3_body(q_ref, b0_ref, g0_ref, bt0_ref, wa1_ref, ba1_ref, ga_ref,
             bta_ref, wa2_ref, ba2_ref, wr1_ref, br1_ref, gr_ref, btr_ref,
             wr2_ref, br2_ref, coords_ref, bsum_ref):
    h = q_ref[:, 0:64] + b0_ref[...]
    h = jax.nn.relu(_ln(h, g0_ref[...], bt0_ref[...]))
    nrm = jnp.sqrt(jnp.sum(h * h, axis=-1, keepdims=True))
    h = h / jnp.maximum(nrm, 1e-12)
    a = jnp.dot(h, wa1_ref[...], preferred_element_type=jnp.float32) + ba1_ref[...]
    a = jax.nn.relu(_ln(a, ga_ref[...], bta_ref[...]))
    th = jnp.dot(a, wa2_ref[...], preferred_element_type=jnp.float32) + ba2_ref[...]
    r = jnp.dot(h, wr1_ref[...], preferred_element_type=jnp.float32) + br1_ref[...]
    r = jax.nn.relu(_ln(r, gr_ref[...], btr_ref[...]))
    rad = jax.nn.sigmoid(
        jnp.dot(r, wr2_ref[...], preferred_element_type=jnp.float32) + br2_ref[...])
    rr = 0.9 + 0.2 * rad
    coords = jnp.concatenate([rr * jnp.cos(th), rr * jnp.sin(th)], axis=1)
    coords_ref[...] = coords
    s = jnp.sum(coords, axis=0, keepdims=True)
    bsum_ref[...] = jnp.concatenate([s, jnp.zeros((7, 2), jnp.float32)], axis=0)


def _k4_body(coords_ref, bsum_ref, out_ref):
    mean = jnp.sum(bsum_ref[...], axis=0, keepdims=True) / _N
    out_ref[...] = coords_ref[...] - mean


@jax.jit
def kernel(x, edge_index, params):
    f32 = jnp.float32
    asf = params["as0"].reshape(1, _HID)
    adf = params["ad0"].reshape(1, _HID)

    ptab = pl.pallas_call(
        _k1_body,
        grid=(_N // _ROWS,),
        in_specs=[
            pl.BlockSpec((_ROWS, _D), lambda i: (i, 0)),
            pl.BlockSpec((_D, _HID), lambda i: (0, 0)),
            pl.BlockSpec((1, _HID), lambda i: (0, 0)),
            pl.BlockSpec((1, _HID), lambda i: (0, 0)),
        ],
        out_specs=pl.BlockSpec((_ROWS, 80), lambda i: (i, 0)),
        out_shape=jax.ShapeDtypeStruct((_N, 80), f32),
    )(x, params["W0"], asf, adf)

    src = edge_index[0].reshape(1, _E)
    dst = edge_index[1].reshape(1, _E)
    qtab = pl.pallas_call(
        _k2_body,
        grid=(3, _E // _C),
        in_specs=[
            pl.BlockSpec((1, _C), lambda p, c: (0, c), memory_space=pltpu.SMEM),
            pl.BlockSpec((1, _C), lambda p, c: (0, c), memory_space=pltpu.SMEM),
            pl.BlockSpec((_N, 80), lambda p, c: (0, 0)),
        ],
        out_specs=pl.BlockSpec((_N, 80), lambda p, c: (0, 0)),
        out_shape=jax.ShapeDtypeStruct((_N, 80), f32),
    )(src, dst, ptab)

    vec = lambda name, w: params[name].reshape(1, w)
    coords_raw, bsums = pl.pallas_call(
        _k3_body,
        grid=(_N // _ROWS,),
        in_specs=[pl.BlockSpec((_ROWS, 80), lambda i: (i, 0))]
        + [pl.BlockSpec(s, lambda i: (0, 0)) for s in
           [(1, 64), (1, 64), (1, 64), (64, 64), (1, 64), (1, 64), (1, 64),
            (64, 1), (1, 1), (64, 32), (1, 32), (1, 32), (1, 32), (32, 1),
            (1, 1)]],
        out_specs=[
            pl.BlockSpec((_ROWS, 2), lambda i: (i, 0)),
            pl.BlockSpec((8, 2), lambda i: (i, 0)),
        ],
        out_shape=[
            jax.ShapeDtypeStruct((_N, 2), f32),
            jax.ShapeDtypeStruct((8 * (_N // _ROWS), 2), f32),
        ],
    )(qtab, vec("b0", 64), vec("g0", 64), vec("bt0", 64), params["Wa1"],
      vec("ba1", 64), vec("ga", 64), vec("bta", 64), params["Wa2"],
      vec("ba2", 1), params["Wr1"], vec("br1", 32), vec("gr", 32),
      vec("btr", 32), params["Wr2"], vec("br2", 1))

    coords = pl.pallas_call(
        _k4_body,
        out_shape=jax.ShapeDtypeStruct((_N, 2), f32),
    )(coords_raw, bsums)
    return coords


# speedup vs baseline: 1.8223x; 1.8223x over previous
import jax
import jax.numpy as jnp
from jax.experimental import pallas as pl
from jax.experimental.pallas import tpu as pltpu
from functools import partial

_N = 50000
_E = 800000
_D = 128
_HID = 64
_H = 8
_HD = 8
_ROWS = 2000
_C = 6400


def _k1_body(x_ref, w_ref, asf_ref, adf_ref, out_ref):
    z = jnp.dot(x_ref[...], w_ref[...], preferred_element_type=jnp.float32)
    lane = jax.lax.broadcasted_iota(jnp.int32, (_HID, _H), 0)
    head = jax.lax.broadcasted_iota(jnp.int32, (_HID, _H), 1)
    g = ((lane // _HD) == head).astype(jnp.float32)
    es = jnp.dot(z * asf_ref[...], g, preferred_element_type=jnp.float32)
    ed = jnp.dot(z * adf_ref[...], g, preferred_element_type=jnp.float32)
    out_ref[...] = jnp.concatenate([z, es, ed], axis=1)


def _leaky(v):
    return jnp.where(v >= 0, v, 0.2 * v)


def _k2_body(src_ref, dst_ref, p_ref, q_ref):
    c = pl.program_id(0)

    @pl.when(c == 0)
    def _init():
        q_ref[:, 0:64] = jnp.zeros((_N, 64), jnp.float32)
        q_ref[:, 64:72] = jnp.full((_N, _H), -jnp.inf, jnp.float32)
        q_ref[:, 72:80] = jnp.zeros((_N, _H), jnp.float32)

    lane = jax.lax.broadcasted_iota(jnp.int32, (_H, _HID), 1)
    head = jax.lax.broadcasted_iota(jnp.int32, (_H, _HID), 0)
    rep = ((lane // _HD) == head).astype(jnp.float32)

    def body(j, _):
        s = src_ref[0, j]
        d = dst_ref[0, j]
        prow = p_ref[pl.ds(s, 1), :]
        ed = p_ref[pl.ds(d, 1), 72:80]
        e8 = _leaky(prow[:, 64:72] + ed)
        qrow = q_ref[pl.ds(d, 1), :]
        m = qrow[:, 64:72]
        mn = jnp.maximum(m, e8)
        sc = jnp.exp(m - mn)
        ee = jnp.exp(e8 - mn)
        den = qrow[:, 72:80] * sc + ee
        sc64 = jnp.dot(sc, rep, preferred_element_type=jnp.float32)
        ee64 = jnp.dot(ee, rep, preferred_element_type=jnp.float32)
        out = qrow[:, 0:64] * sc64 + prow[:, 0:64] * ee64
        q_ref[pl.ds(d, 1), :] = jnp.concatenate([out, mn, den], axis=1)
        return 0

    jax.lax.fori_loop(0, _C, body, 0)


def _ln(v, g, b):
    m = jnp.mean(v, axis=-1, keepdims=True)
    var = jnp.mean((v - m) * (v - m), axis=-1, keepdims=True)
    return (v - m) * jax.lax.rsqrt(var + 1e-5) * g + b


def _k3_body(q_ref, b0_ref, g0_ref, bt0_ref, wa1_ref, ba1_ref, ga_ref,
             bta_ref, wa2_ref, ba2_ref, wr1_ref, br1_ref, gr_ref, btr_ref,
             wr2_ref, br2_ref, coords_ref, bsum_ref):
    lane = jax.lax.broadcasted_iota(jnp.int32, (_H, _HID), 1)
    head = jax.lax.broadcasted_iota(jnp.int32, (_H, _HID), 0)
    rep = ((lane // _HD) == head).astype(jnp.float32)
    den64 = jnp.dot(q_ref[:, 72:80], rep, preferred_element_type=jnp.float32)
    h = q_ref[:, 0:64] / (den64 + 1e-16) + b0_ref[...]
    h = jax.nn.relu(_ln(h, g0_ref[...], bt0_ref[...]))
    nrm = jnp.sqrt(jnp.sum(h * h, axis=-1, keepdims=True))
    h = h / jnp.maximum(nrm, 1e-12)
    a = jnp.dot(h, wa1_ref[...], preferred_element_type=jnp.float32) + ba1_ref[...]
    a = jax.nn.relu(_ln(a, ga_ref[...], bta_ref[...]))
    th = jnp.dot(a, wa2_ref[...], preferred_element_type=jnp.float32) + ba2_ref[...]
    r = jnp.dot(h, wr1_ref[...], preferred_element_type=jnp.float32) + br1_ref[...]
    r = jax.nn.relu(_ln(r, gr_ref[...], btr_ref[...]))
    rad = jax.nn.sigmoid(
        jnp.dot(r, wr2_ref[...], preferred_element_type=jnp.float32) + br2_ref[...])
    rr = 0.9 + 0.2 * rad
    coords = jnp.concatenate([rr * jnp.cos(th), rr * jnp.sin(th)], axis=1)
    coords_ref[...] = coords
    s = jnp.sum(coords, axis=0, keepdims=True)
    bsum_ref[...] = jnp.concatenate([s, jnp.zeros((7, 2), jnp.float32)], axis=0)


def _k4_body(coords_ref, bsum_ref, out_ref):
    mean = jnp.sum(bsum_ref[...], axis=0, keepdims=True) / _N
    out_ref[...] = coords_ref[...] - mean


@jax.jit
def kernel(x, edge_index, params):
    f32 = jnp.float32
    asf = params["as0"].reshape(1, _HID)
    adf = params["ad0"].reshape(1, _HID)

    ptab = pl.pallas_call(
        _k1_body,
        grid=(_N // _ROWS,),
        in_specs=[
            pl.BlockSpec((_ROWS, _D), lambda i: (i, 0)),
            pl.BlockSpec((_D, _HID), lambda i: (0, 0)),
            pl.BlockSpec((1, _HID), lambda i: (0, 0)),
            pl.BlockSpec((1, _HID), lambda i: (0, 0)),
        ],
        out_specs=pl.BlockSpec((_ROWS, 80), lambda i: (i, 0)),
        out_shape=jax.ShapeDtypeStruct((_N, 80), f32),
    )(x, params["W0"], asf, adf)

    src = edge_index[0].reshape(1, _E)
    dst = edge_index[1].reshape(1, _E)
    qtab = pl.pallas_call(
        _k2_body,
        grid=(_E // _C,),
        in_specs=[
            pl.BlockSpec((1, _C), lambda c: (0, c), memory_space=pltpu.SMEM),
            pl.BlockSpec((1, _C), lambda c: (0, c), memory_space=pltpu.SMEM),
            pl.BlockSpec((_N, 80), lambda c: (0, 0)),
        ],
        out_specs=pl.BlockSpec((_N, 80), lambda c: (0, 0)),
        out_shape=jax.ShapeDtypeStruct((_N, 80), f32),
    )(src, dst, ptab)

    vec = lambda name, w: params[name].reshape(1, w)
    coords_raw, bsums = pl.pallas_call(
        _k3_body,
        grid=(_N // _ROWS,),
        in_specs=[pl.BlockSpec((_ROWS, 80), lambda i: (i, 0))]
        + [pl.BlockSpec(s, lambda i: (0, 0)) for s in
           [(1, 64), (1, 64), (1, 64), (64, 64), (1, 64), (1, 64), (1, 64),
            (64, 1), (1, 1), (64, 32), (1, 32), (1, 32), (1, 32), (32, 1),
            (1, 1)]],
        out_specs=[
            pl.BlockSpec((_ROWS, 2), lambda i: (i, 0)),
            pl.BlockSpec((8, 2), lambda i: (i, 0)),
        ],
        out_shape=[
            jax.ShapeDtypeStruct((_N, 2), f32),
            jax.ShapeDtypeStruct((8 * (_N // _ROWS), 2), f32),
        ],
    )(qtab, vec("b0", 64), vec("g0", 64), vec("bt0", 64), params["Wa1"],
      vec("ba1", 64), vec("ga", 64), vec("bta", 64), params["Wa2"],
      vec("ba2", 1), params["Wr1"], vec("br1", 32), vec("gr", 32),
      vec("btr", 32), params["Wr2"], vec("br2", 1))

    coords = pl.pallas_call(
        _k4_body,
        out_shape=jax.ShapeDtypeStruct((_N, 2), f32),
    )(coords_raw, bsums)
    return coords
